# Initial kernel scaffold; baseline (speedup 1.0000x reference)
import jax
import jax.numpy as jnp
from jax.experimental import pallas as pl


def kernel(pos_enc, pe):
    raise NotImplementedError("write your pallas kernel here")


# speedup vs baseline: 3.6052x; 3.6052x over previous
import functools

import jax
import jax.numpy as jnp
from jax import lax
from jax.experimental import pallas as pl
from jax.experimental.pallas import tpu as pltpu
from jax.experimental.pallas import tpu_sc as plsc

N = 327680
D_MODEL = 256
POS_MAX = 128
D = D_MODEL // 4

NC, NS = 2, 16
NW = NC * NS
B = N * 4
B_PER_W = B // NW

CHUNK = 128
NCHUNK = B_PER_W // CHUNK
NBUF = 2
GPB = 4
ROWS_PER_BUF = GPB * CHUNK
ROUNDS = NCHUNK // (NBUF * GPB)


def _body(pe_hbm, idx_hbm, out_hbm, idx_v, rows_v, *sems):
    gsem = sems[:NBUF]
    osem = sems[NBUF:]
    wid = lax.axis_index("s") * NC + lax.axis_index("c")

    pltpu.sync_copy(idx_hbm.at[wid], idx_v)
    out_base = wid * B_PER_W

    def fire(r, b):
        for g in range(GPB):
            j = (r * NBUF + b) * GPB + g
            pltpu.async_copy(
                pe_hbm.at[idx_v.at[j]],
                rows_v.at[b].at[pl.ds(g * CHUNK, CHUNK)],
                gsem[b],
            )

    def drain(r, b):
        for g in range(GPB):
            j = (r * NBUF + b) * GPB + g
            pltpu.make_async_copy(
                pe_hbm.at[idx_v.at[j]],
                rows_v.at[b].at[pl.ds(g * CHUNK, CHUNK)],
                gsem[b],
            ).wait()

    def out_slice(r, b):
        start = out_base + (r * NBUF + b) * ROWS_PER_BUF
        return out_hbm.at[pl.ds(start, ROWS_PER_BUF)]

    def scatter_start(r, b):
        pltpu.async_copy(rows_v.at[b], out_slice(r, b), osem[b])

    def scatter_wait(r, b):
        pltpu.make_async_copy(rows_v.at[b], out_slice(r, b), osem[b]).wait()

    for b in range(NBUF):
        fire(0, b)

    def round_body(r, _):
        for b in range(NBUF):
            drain(r, b)
            scatter_start(r, b)
            scatter_wait(r, b)
            fire(r + 1, b)
        return _

    lax.fori_loop(0, ROUNDS - 1, round_body, 0, unroll=False)

    for b in range(NBUF):
        drain(ROUNDS - 1, b)
        scatter_start(ROUNDS - 1, b)
        scatter_wait(ROUNDS - 1, b)


@jax.jit
def _gather_all(pe, idx3):
    mesh = plsc.VectorSubcoreMesh(
        core_axis_name="c", subcore_axis_name="s", num_cores=NC, num_subcores=NS
    )
    scratch = [
        pltpu.VMEM((NCHUNK, CHUNK), jnp.int32),
        pltpu.VMEM((NBUF, ROWS_PER_BUF, D), jnp.float32),
    ] + [pltpu.SemaphoreType.DMA] * (2 * NBUF)
    return pl.kernel(
        _body,
        out_type=jax.ShapeDtypeStruct((B, D), jnp.float32),
        mesh=mesh,
        scratch_types=scratch,
        compiler_params=pltpu.CompilerParams(use_tc_tiling_on_sc=False),
    )(pe, idx3)


def kernel(pos_enc, pe):
    idx3 = pos_enc.astype(jnp.int32).reshape(NW, NCHUNK, CHUNK)
    out2 = _gather_all(pe, idx3)
    return out2.reshape(N, D_MODEL)


# speedup vs baseline: 7.0934x; 1.9676x over previous
import functools

import jax
import jax.numpy as jnp
from jax import lax
from jax.experimental import pallas as pl
from jax.experimental.pallas import tpu as pltpu
from jax.experimental.pallas import tpu_sc as plsc

N = 327680
D_MODEL = 256
POS_MAX = 128
D = D_MODEL // 4

NC, NS = 2, 16
NW = NC * NS
B = N * 4
B_PER_W = B // NW

CHUNK = 128
NCHUNK = B_PER_W // CHUNK
NBUF = 2
GPB = 4
ROWS_PER_BUF = GPB * CHUNK
ROUNDS = NCHUNK // (NBUF * GPB)


def _body(pe_hbm, idx_hbm, out_hbm, pe_sh, idx_v, rows_v, *sems):
    gsem = sems[:NBUF]
    osem = sems[NBUF:]
    sid = lax.axis_index("s")
    wid = sid * NC + lax.axis_index("c")

    @pl.when(sid == 0)
    def _stage_table():
        tbl_stage = rows_v.at[0].at[pl.ds(0, POS_MAX)]
        pltpu.sync_copy(pe_hbm, tbl_stage)
        pltpu.sync_copy(tbl_stage, pe_sh)

    pltpu.sync_copy(idx_hbm.at[wid], idx_v)
    plsc.subcore_barrier()
    out_base = wid * B_PER_W

    def fire(r, b):
        for g in range(GPB):
            j = (r * NBUF + b) * GPB + g
            pltpu.async_copy(
                pe_sh.at[idx_v.at[j]],
                rows_v.at[b].at[pl.ds(g * CHUNK, CHUNK)],
                gsem[b],
            )

    def drain(r, b):
        for g in range(GPB):
            j = (r * NBUF + b) * GPB + g
            pltpu.make_async_copy(
                pe_sh.at[idx_v.at[j]],
                rows_v.at[b].at[pl.ds(g * CHUNK, CHUNK)],
                gsem[b],
            ).wait()

    def out_slice(r, b):
        start = out_base + (r * NBUF + b) * ROWS_PER_BUF
        return out_hbm.at[pl.ds(start, ROWS_PER_BUF)]

    def scatter_start(r, b):
        pltpu.async_copy(rows_v.at[b], out_slice(r, b), osem[b])

    def scatter_wait(r, b):
        pltpu.make_async_copy(rows_v.at[b], out_slice(r, b), osem[b]).wait()

    for b in range(NBUF):
        fire(0, b)

    def round_body(r, _):
        for b in range(NBUF):
            drain(r, b)
            scatter_start(r, b)
            scatter_wait(r, b)
            fire(r + 1, b)
        return _

    lax.fori_loop(0, ROUNDS - 1, round_body, 0, unroll=False)

    for b in range(NBUF):
        drain(ROUNDS - 1, b)
        scatter_start(ROUNDS - 1, b)
        scatter_wait(ROUNDS - 1, b)


@jax.jit
def _gather_all(pe, idx3):
    mesh = plsc.VectorSubcoreMesh(
        core_axis_name="c", subcore_axis_name="s", num_cores=NC, num_subcores=NS
    )
    scratch = [
        pltpu.VMEM_SHARED((POS_MAX, D), jnp.float32),
        pltpu.VMEM((NCHUNK, CHUNK), jnp.int32),
        pltpu.VMEM((NBUF, ROWS_PER_BUF, D), jnp.float32),
    ] + [pltpu.SemaphoreType.DMA] * (2 * NBUF)
    return pl.kernel(
        _body,
        out_type=jax.ShapeDtypeStruct((B, D), jnp.float32),
        mesh=mesh,
        scratch_types=scratch,
        compiler_params=pltpu.CompilerParams(use_tc_tiling_on_sc=False),
    )(pe, idx3)


def kernel(pos_enc, pe):
    idx3 = pos_enc.astype(jnp.int32).reshape(NW, NCHUNK, CHUNK)
    out2 = _gather_all(pe, idx3)
    return out2.reshape(N, D_MODEL)


# speedup vs baseline: 7.1010x; 1.0011x over previous
import functools

import jax
import jax.numpy as jnp
from jax import lax
from jax.experimental import pallas as pl
from jax.experimental.pallas import tpu as pltpu
from jax.experimental.pallas import tpu_sc as plsc

N = 327680
D_MODEL = 256
POS_MAX = 128
D = D_MODEL // 4

NC, NS = 2, 16
NW = NC * NS
OUT_PER_W = N // NW
B_PER_W = OUT_PER_W * 4

CHUNK = 128
NCHUNK = B_PER_W // CHUNK
NBUF = 2
GPB = 4
ROWS_PER_BUF = GPB * CHUNK
OUT_PER_BUF = ROWS_PER_BUF // 4
ROUNDS = NCHUNK // (NBUF * GPB)


def _body(pe_hbm, idx_hbm, out_hbm, pe_sh, idx_v, rows_v, *sems):
    gsem = sems[:NBUF]
    osem = sems[NBUF:]
    sid = lax.axis_index("s")
    wid = sid * NC + lax.axis_index("c")

    @pl.when(sid == 0)
    def _stage_table():
        tbl_stage = rows_v.at[0].at[pl.ds(0, POS_MAX)]
        pltpu.sync_copy(pe_hbm, tbl_stage)
        pltpu.sync_copy(tbl_stage, pe_sh)

    row0 = wid * OUT_PER_W
    pltpu.sync_copy(idx_hbm.at[wid], idx_v)
    idx_c = idx_v
    plsc.subcore_barrier()

    def fire(r, b):
        for g in range(GPB):
            j = (r * NBUF + b) * GPB + g
            pltpu.async_copy(
                pe_sh.at[idx_c.at[j]],
                rows_v.at[b].at[pl.ds(g * CHUNK, CHUNK)],
                gsem[b],
            )

    def drain(r, b):
        for g in range(GPB):
            j = (r * NBUF + b) * GPB + g
            pltpu.make_async_copy(
                pe_sh.at[idx_c.at[j]],
                rows_v.at[b].at[pl.ds(g * CHUNK, CHUNK)],
                gsem[b],
            ).wait()

    def out_slice(r, b):
        start = 4 * row0 + (r * NBUF + b) * ROWS_PER_BUF
        return out_hbm.at[pl.ds(start, ROWS_PER_BUF)]

    def buf_as_out(b):
        return rows_v.at[b]

    def scatter_start(r, b):
        pltpu.async_copy(buf_as_out(b), out_slice(r, b), osem[b])

    def scatter_wait(r, b):
        pltpu.make_async_copy(buf_as_out(b), out_slice(r, b), osem[b]).wait()

    for b in range(NBUF):
        fire(0, b)

    def round_body(r, _):
        for b in range(NBUF):
            drain(r, b)
            scatter_start(r, b)
            scatter_wait(r, b)
            fire(r + 1, b)
        return _

    lax.fori_loop(0, ROUNDS - 1, round_body, 0, unroll=False)

    for b in range(NBUF):
        drain(ROUNDS - 1, b)
        scatter_start(ROUNDS - 1, b)
        scatter_wait(ROUNDS - 1, b)


@jax.jit
def _gather_all(pe, idx):
    mesh = plsc.VectorSubcoreMesh(
        core_axis_name="c", subcore_axis_name="s", num_cores=NC, num_subcores=NS
    )
    scratch = [
        pltpu.VMEM_SHARED((POS_MAX, D), jnp.float32),
        pltpu.VMEM((NCHUNK, CHUNK), jnp.int32),
        pltpu.VMEM((NBUF, ROWS_PER_BUF, D), jnp.float32),
    ] + [pltpu.SemaphoreType.DMA] * (2 * NBUF)
    out = pl.kernel(
        _body,
        out_type=jax.ShapeDtypeStruct((N * 4, D), jnp.float32),
        mesh=mesh,
        scratch_types=scratch,
        compiler_params=pltpu.CompilerParams(use_tc_tiling_on_sc=False),
    )(pe, idx)
    return out.reshape(N, D_MODEL)


def kernel(pos_enc, pe):
    idx = pos_enc.astype(jnp.int32).reshape(NW, NCHUNK, CHUNK)
    return _gather_all(pe, idx)
